# baseline (speedup 1.0000x reference)
import functools

import jax
import jax.numpy as jnp
from jax import lax
from jax.experimental import pallas as pl
from jax.experimental.pallas import tpu as pltpu
from jax.experimental.pallas import tpu_sc as plsc

_B = 320000
_D = 128
_L = 16
_NC = 2
_NS = 16
_NW = _NC * _NS
_PER_W = _B // _NW
_C = 80
_NCHUNK = _PER_W // _C
_NBUF = 3


def _tec_body(z_hbm, ei_hbm, out_hbm,
              idx_f, rows_t0, rows_t1, rows_t2, res_f, part_v,
              s0, s1, s2):
    wid = lax.axis_index("s") * _NC + lax.axis_index("c")
    base = wid * _PER_W

    pltpu.sync_copy(ei_hbm.at[pl.ds(wid * 2 * _PER_W, 2 * _PER_W)], idx_f)

    bufs = (rows_t0, rows_t1, rows_t2)
    sems = (s0, s1, s2)
    lanes = lax.iota(jnp.int32, _L)
    lanes17 = lanes * 17

    def issue(i, buf, sem):
        pltpu.async_copy(z_hbm.at[idx_f.at[pl.ds(i * 2 * _C, 2 * _C)]], buf, sem)

    def wait(i, buf, sem):
        pltpu.make_async_copy(
            z_hbm.at[idx_f.at[pl.ds(i * 2 * _C, 2 * _C)]], buf, sem).wait()

    def compute(i, rt):
        def group(g, _):
            base_c = g * _L
            for e2 in range(_L):
                c = base_c + e2
                s = [rt[c, pl.ds(l * _L, _L)] * rt[_C + c, pl.ds(l * _L, _L)]
                     for l in range(_D // _L)]
                acc = (((s[0] + s[1]) + (s[2] + s[3]))
                       + ((s[4] + s[5]) + (s[6] + s[7])))
                part_v[pl.ds(e2 * 17, _L)] = acc
            t = [plsc.load_gather(part_v, [lanes17 + l]) for l in range(_L)]
            t = [t[2 * k] + t[2 * k + 1] for k in range(8)]
            t = [t[2 * k] + t[2 * k + 1] for k in range(4)]
            tot = (t[0] + t[1]) + (t[2] + t[3])
            res_f[pl.ds(i * _C + base_c, _L)] = 1.0 / (1.0 + jnp.exp(-tot))
            return ()

        lax.fori_loop(0, _C // _L, group, ())

    for r in range(_NBUF):
        issue(r, bufs[r], sems[r])

    def body(j, _):
        for r in range(_NBUF):
            i = _NBUF * j + r
            wait(i, bufs[r], sems[r])
            compute(i, bufs[r])

            @pl.when(i + _NBUF < _NCHUNK)
            def _():
                issue(i + _NBUF, bufs[r], sems[r])

        return ()

    lax.fori_loop(0, _NCHUNK // _NBUF, body, ())

    for r in range(_NCHUNK % _NBUF):
        i = (_NCHUNK // _NBUF) * _NBUF + r
        wait(i, bufs[r], sems[r])
        compute(i, bufs[r])

    pltpu.sync_copy(res_f, out_hbm.at[pl.ds(base, _PER_W)])


@functools.partial(jax.jit, static_argnums=())
def _sc_call(z, ei):
    mesh = plsc.VectorSubcoreMesh(core_axis_name="c", subcore_axis_name="s")
    f = pl.kernel(
        _tec_body,
        mesh=mesh,
        compiler_params=pltpu.CompilerParams(needs_layout_passes=False),
        out_type=jax.ShapeDtypeStruct((_B,), jnp.float32),
        scratch_types=[
            pltpu.VMEM((2 * _PER_W,), jnp.int32),
            pltpu.VMEM((2 * _C, _D), jnp.float32),
            pltpu.VMEM((2 * _C, _D), jnp.float32),
            pltpu.VMEM((2 * _C, _D), jnp.float32),
            pltpu.VMEM((_PER_W,), jnp.float32),
            pltpu.VMEM((_L * 17,), jnp.float32),
            pltpu.SemaphoreType.DMA,
            pltpu.SemaphoreType.DMA,
            pltpu.SemaphoreType.DMA,
        ],
    )
    return f(z, ei)


def kernel(z, e):
    ei = (e.astype(jnp.int32)
          .reshape(2, _NW, _NCHUNK, _C)
          .transpose(1, 2, 0, 3)
          .reshape(-1))
    return _sc_call(z, ei)
